# baseline (speedup 1.0000x reference)
import functools

import jax
import jax.numpy as jnp
from jax import lax
from jax.experimental import pallas as pl
from jax.experimental.pallas import tpu as pltpu
from jax.experimental.pallas import tpu_sc as plsc

N = 50000
E = 800000
F_IN = 128
H = 64
C = 6
HQ = H // 4

R2 = 6400
E_PAD = R2 * 128
ACC_ROWS = 51200
STRIPE = ACC_ROWS // 16
WCHUNK = 320

ROWS_T = R2 // 16
K = 8
SUPERS = ROWS_T // K

DEG_ROWS_T = R2 // 2 // 16
K_DEG = 8
DEG_SUPERS = DEG_ROWS_T // K_DEG

BR = 2048
NB = -(-N // BR)
FB = BR * H // 128
QFB = BR * HQ // 128
QF = N * HQ // 128

_mesh = plsc.VectorSubcoreMesh(core_axis_name="c", subcore_axis_name="s")
_sc_params = pltpu.CompilerParams(use_tc_tiling_on_sc=False)


@functools.partial(
    pl.kernel,
    mesh=_mesh,
    compiler_params=_sc_params,
    out_type=jax.ShapeDtypeStruct((2 * ACC_ROWS,), jnp.float32),
    scratch_types=[
        pltpu.VMEM((K_DEG, 128), jnp.int32),
        pltpu.VMEM((128,), jnp.float32),
        pltpu.VMEM((STRIPE,), jnp.float32),
        pltpu.VMEM_SHARED((ACC_ROWS,), jnp.float32),
        pltpu.SemaphoreType.DMA,
    ],
)
def _deg_kernel(dst_hbm, zeros_hbm, ones_hbm, out_hbm, idx_v, ones_v, wbuf, acc, sem):
    c = lax.axis_index("c")
    s = lax.axis_index("s")
    pltpu.sync_copy(zeros_hbm, wbuf)
    pltpu.sync_copy(wbuf, acc.at[pl.ds(s * STRIPE, STRIPE)])
    pltpu.sync_copy(ones_hbm, ones_v)
    plsc.subcore_barrier()

    row0 = c * (R2 // 2) + s * DEG_ROWS_T

    def step(u, carry):
        r0 = row0 + u * K_DEG
        pltpu.sync_copy(dst_hbm.at[pl.ds(r0, K_DEG)], idx_v)
        for j in range(K_DEG):
            pltpu.sync_copy(ones_v, acc.at[idx_v.at[j]], add=True)
        return carry

    lax.fori_loop(0, DEG_SUPERS, step, 0)
    plsc.subcore_barrier()

    pltpu.sync_copy(acc.at[pl.ds(s * STRIPE, STRIPE)], wbuf)
    pltpu.sync_copy(wbuf, out_hbm.at[pl.ds(c * ACC_ROWS + s * STRIPE, STRIPE)])


@functools.partial(
    pl.kernel,
    mesh=_mesh,
    compiler_params=_sc_params,
    out_type=jax.ShapeDtypeStruct((ACC_ROWS, H), jnp.float32),
    scratch_types=[
        pltpu.VMEM((K, 128), jnp.int32),
        pltpu.VMEM((K, 128), jnp.int32),
        pltpu.VMEM((K, 128), jnp.int32),
        pltpu.VMEM((K, 128), jnp.int32),
        pltpu.VMEM((K, 128, HQ), jnp.float32),
        pltpu.VMEM((K, 128, HQ), jnp.float32),
        pltpu.VMEM((WCHUNK, HQ), jnp.float32),
        pltpu.VMEM_SHARED((ACC_ROWS, HQ), jnp.float32),
        pltpu.SemaphoreType.DMA,
        pltpu.SemaphoreType.DMA,
        pltpu.SemaphoreType.DMA,
        pltpu.SemaphoreType.DMA,
    ],
)
def _gs_kernel(t0_hbm, t1_hbm, t2_hbm, t3_hbm, src_hbm, dst_hbm, zeros_hbm,
               out_hbm,
               idx_sA, idx_dA, idx_sB, idx_dB, rowsA, rowsB, wbuf, acc,
               gsemA, gsemB, ssemA, ssemB):
    c = lax.axis_index("c")
    s = lax.axis_index("s")
    row0 = s * ROWS_T

    def run(q, ts_hbm):
        pltpu.sync_copy(zeros_hbm, wbuf)
        for m in range(STRIPE // WCHUNK):
            pltpu.sync_copy(wbuf, acc.at[pl.ds(s * STRIPE + m * WCHUNK, WCHUNK)])
        plsc.subcore_barrier()

        col = q * HQ

        def drain_gathers(rows, gsem):
            for j in range(K):
                pltpu.make_async_copy(
                    ts_hbm.at[pl.ds(0, 128)], rows.at[j], gsem).wait()

        def scatter(idx_d, rows, ssem):
            for j in range(K):
                pltpu.async_copy(rows.at[j], acc.at[idx_d.at[j]], ssem,
                                 add=True)

        def drain_scatters(idx_d, rows, ssem):
            for j in range(K):
                pltpu.make_async_copy(rows.at[j], acc.at[idx_d.at[j]],
                                      ssem).wait()

        def half(u, idx_s, idx_d, rows, gsem, ssem, pidx_d, prows, pgsem, pssem):
            @pl.when(u >= 2)
            def _():
                drain_scatters(idx_d, rows, ssem)

            r0 = row0 + u * K
            pltpu.sync_copy(src_hbm.at[pl.ds(r0, K)], idx_s)
            pltpu.sync_copy(dst_hbm.at[pl.ds(r0, K)], idx_d)
            for j in range(K):
                pltpu.async_copy(ts_hbm.at[idx_s.at[j]], rows.at[j], gsem)

            @pl.when(u >= 1)
            def _():
                drain_gathers(prows, pgsem)
                scatter(pidx_d, prows, pssem)

        def step(u, carry):
            @pl.when(u % 2 == 0)
            def _():
                half(u, idx_sA, idx_dA, rowsA, gsemA, ssemA,
                     idx_dB, rowsB, gsemB, ssemB)

            @pl.when(u % 2 == 1)
            def _():
                half(u, idx_sB, idx_dB, rowsB, gsemB, ssemB,
                     idx_dA, rowsA, gsemA, ssemA)

            return carry

        lax.fori_loop(0, SUPERS, step, 0)
        drain_gathers(rowsB, gsemB)
        scatter(idx_dB, rowsB, ssemB)
        drain_scatters(idx_dA, rowsA, ssemA)
        drain_scatters(idx_dB, rowsB, ssemB)
        plsc.subcore_barrier()
        for m in range(STRIPE // WCHUNK):
            off = s * STRIPE + m * WCHUNK
            pltpu.sync_copy(acc.at[pl.ds(off, WCHUNK)], wbuf)
            pltpu.sync_copy(wbuf, out_hbm.at[pl.ds(off, WCHUNK), pl.ds(col, HQ)])
        plsc.subcore_barrier()

    @pl.when(c == 0)
    def _():
        run(0, t0_hbm)
        run(1, t1_hbm)

    @pl.when(c == 1)
    def _():
        run(2, t2_hbm)
        run(3, t3_hbm)


def _quarters(t):
    return [t[:, q * HQ:(q + 1) * HQ] for q in range(4)]


def _t1_body(x_ref, deg_ref, embW_ref, embb_ref, W1_ref,
             t0_ref, t1_ref, t2_ref, t3_ref):
    h0 = jnp.dot(x_ref[...], embW_ref[...], preferred_element_type=jnp.float32)
    h0 = jnp.maximum(h0 + embb_ref[...][None, :], 0.0)
    dinv = lax.rsqrt(deg_ref[...] + 1.0)
    t = jnp.dot(h0 * dinv, W1_ref[...], preferred_element_type=jnp.float32)
    for ref, q in zip((t0_ref, t1_ref, t2_ref, t3_ref), _quarters(t)):
        ref[...] = q


_t1_call = pl.pallas_call(
    _t1_body,
    grid=(NB,),
    in_specs=[
        pl.BlockSpec((BR, F_IN), lambda i: (i, 0)),
        pl.BlockSpec((BR, 1), lambda i: (i, 0)),
        pl.BlockSpec((F_IN, H), lambda i: (0, 0)),
        pl.BlockSpec((H,), lambda i: (0,)),
        pl.BlockSpec((H, H), lambda i: (0, 0)),
    ],
    out_specs=[pl.BlockSpec((BR, HQ), lambda i: (i, 0))] * 4,
    out_shape=[jax.ShapeDtypeStruct((N, HQ), jnp.float32)] * 4,
)


def _m1_body(s_ref, t0_ref, t1_ref, t2_ref, t3_ref, deg_ref, b_ref,
             g_ref, be_ref, W_ref, o0_ref, o1_ref, o2_ref, o3_ref,
             zscr, stscr):
    def phase1(h_pre, dinv):
        h = jnp.maximum(h_pre * g_ref[...][None, :] + be_ref[...][None, :], 0.0)
        t = jnp.dot(h * dinv, W_ref[...], preferred_element_type=jnp.float32)
        for ref, q in zip((o0_ref, o1_ref, o2_ref, o3_ref), _quarters(t)):
            ref[...] = q

    _gcn_epilogue_run(s_ref, t0_ref, t1_ref, t2_ref, t3_ref, deg_ref, b_ref,
                      zscr, stscr, phase1)


def _gcn_epilogue_run(s_ref, t0_ref, t1_ref, t2_ref, t3_ref, deg_ref, b_ref,
                      zscr, stscr, phase1):
    p = pl.program_id(0)
    i = pl.program_id(1)

    @pl.when(p == 0)
    def _():
        S = s_ref[...]
        ts = jnp.concatenate(
            [r[...] for r in (t0_ref, t1_ref, t2_ref, t3_ref)], axis=1)
        dinv = lax.rsqrt(deg_ref[...] + 1.0)
        z = dinv * (S + ts) + b_ref[...][None, :]
        zscr[pl.ds(i * BR, BR), :] = z

        @pl.when(i == 0)
        def _():
            stscr[...] = jnp.zeros_like(stscr)

        rowid = i * BR + lax.broadcasted_iota(jnp.int32, (BR, 1), 0)
        zm = jnp.where(rowid < N, z, 0.0)
        stscr[...] += jnp.stack([jnp.sum(zm, axis=0), jnp.sum(zm * zm, axis=0)])

    @pl.when(p == 1)
    def _():
        z = zscr[pl.ds(i * BR, BR), :]
        st = stscr[...]
        mu = st[0:1, :] * (1.0 / N)
        var = st[1:2, :] * (1.0 / N) - mu * mu
        rstd = lax.rsqrt(var + 1e-5)
        dinv = lax.rsqrt(deg_ref[...] + 1.0)
        phase1((z - mu) * rstd, dinv)


def _m2_body(s_ref, t0_ref, t1_ref, t2_ref, t3_ref, deg_ref, b_ref,
             g_ref, be_ref, clsW_ref, clsb_ref, out_ref, zscr, stscr):
    def phase1(h_pre, dinv):
        h = jnp.maximum(h_pre * g_ref[...][None, :] + be_ref[...][None, :], 0.0)
        out_ref[...] = (
            jnp.dot(h, clsW_ref[...], preferred_element_type=jnp.float32)
            + clsb_ref[...][None, :])

    _gcn_epilogue_run(s_ref, t0_ref, t1_ref, t2_ref, t3_ref, deg_ref, b_ref,
                      zscr, stscr, phase1)


def _epilogue_in_specs():
    return [
        pl.BlockSpec((BR, H), lambda p, i: ((1 - p) * i, 0)),
        pl.BlockSpec((BR, HQ), lambda p, i: ((1 - p) * i, 0)),
        pl.BlockSpec((BR, HQ), lambda p, i: ((1 - p) * i, 0)),
        pl.BlockSpec((BR, HQ), lambda p, i: ((1 - p) * i, 0)),
        pl.BlockSpec((BR, HQ), lambda p, i: ((1 - p) * i, 0)),
        pl.BlockSpec((BR, 1), lambda p, i: (i, 0)),
        pl.BlockSpec((H,), lambda p, i: (0,)),
        pl.BlockSpec((H,), lambda p, i: (0,)),
        pl.BlockSpec((H,), lambda p, i: (0,)),
    ]


_m1_call = pl.pallas_call(
    _m1_body,
    grid=(2, NB),
    in_specs=_epilogue_in_specs() + [
        pl.BlockSpec((H, H), lambda p, i: (0, 0)),
    ],
    out_specs=[pl.BlockSpec((BR, HQ), lambda p, i: (i, 0))] * 4,
    out_shape=[jax.ShapeDtypeStruct((N, HQ), jnp.float32)] * 4,
    scratch_shapes=[
        pltpu.VMEM((NB * BR, H), jnp.float32),
        pltpu.VMEM((2, H), jnp.float32),
    ],
)

_m2_call = pl.pallas_call(
    _m2_body,
    grid=(2, NB),
    in_specs=_epilogue_in_specs() + [
        pl.BlockSpec((H, C), lambda p, i: (0, 0)),
        pl.BlockSpec((C,), lambda p, i: (0,)),
    ],
    out_specs=pl.BlockSpec((BR, C), lambda p, i: (i, 0)),
    out_shape=jax.ShapeDtypeStruct((N, C), jnp.float32),
    scratch_shapes=[
        pltpu.VMEM((NB * BR, H), jnp.float32),
        pltpu.VMEM((2, H), jnp.float32),
    ],
)


def kernel(x, edge_index, emb_W, emb_b, W1, b1, g1, be1, W2, b2, g2, be2, cls_W, cls_b):
    src = edge_index[0]
    dst = edge_index[1]
    pad = E_PAD - E
    pad_idx = jnp.arange(pad, dtype=jnp.int32)
    pad_src = (pad_idx * 97) % N
    pad_dst = N + (pad_idx % (ACC_ROWS - N))
    src2d = jnp.concatenate([src, pad_src]).reshape(R2, 128)
    dst2d = jnp.concatenate([dst, pad_dst]).reshape(R2, 128)

    zeros1d = jnp.zeros((STRIPE,), jnp.float32)
    zeros2d = jnp.zeros((WCHUNK, HQ), jnp.float32)
    ones128 = jnp.ones((128,), jnp.float32)

    degp = _deg_kernel(dst2d, zeros1d, ones128)
    deg_col = (degp[:N] + degp[ACC_ROWS:ACC_ROWS + N]).reshape(N, 1)

    ts1 = _t1_call(x, deg_col, emb_W, emb_b, W1)
    S1 = _gs_kernel(*ts1, src2d, dst2d, zeros2d)
    ts2 = _m1_call(S1, *ts1, deg_col, b1, g1, be1, W2)
    S2 = _gs_kernel(*ts2, src2d, dst2d, zeros2d)
    return _m2_call(S2, *ts2, deg_col, b2, g2, be2, cls_W, cls_b)


# speedup vs baseline: 1.0212x; 1.0212x over previous
import functools

import jax
import jax.numpy as jnp
from jax import lax
from jax.experimental import pallas as pl
from jax.experimental.pallas import tpu as pltpu
from jax.experimental.pallas import tpu_sc as plsc

N = 50000
E = 800000
F_IN = 128
H = 64
C = 6
HQ = H // 4

R2 = 6400
E_PAD = R2 * 128
ACC_ROWS = 51200
STRIPE = ACC_ROWS // 16
WCHUNK = 320

ROWS_T = R2 // 16
K = 8
SUPERS = ROWS_T // K

DEG_ROWS_T = R2 // 2 // 16
K_DEG = 8
DEG_SUPERS = DEG_ROWS_T // K_DEG

BR = 2048
NB = -(-N // BR)
FB = BR * H // 128
QFB = BR * HQ // 128
QF = N * HQ // 128

_mesh = plsc.VectorSubcoreMesh(core_axis_name="c", subcore_axis_name="s")
_sc_params = pltpu.CompilerParams(use_tc_tiling_on_sc=False)


@functools.partial(
    pl.kernel,
    mesh=_mesh,
    compiler_params=_sc_params,
    out_type=jax.ShapeDtypeStruct((2 * ACC_ROWS,), jnp.float32),
    scratch_types=[
        pltpu.VMEM((K_DEG, 128), jnp.int32),
        pltpu.VMEM((128,), jnp.float32),
        pltpu.VMEM((STRIPE,), jnp.float32),
        pltpu.VMEM_SHARED((ACC_ROWS,), jnp.float32),
        pltpu.SemaphoreType.DMA,
    ],
)
def _deg_kernel(dst_hbm, zeros_hbm, ones_hbm, out_hbm, idx_v, ones_v, wbuf, acc, sem):
    c = lax.axis_index("c")
    s = lax.axis_index("s")
    pltpu.sync_copy(zeros_hbm, wbuf)
    pltpu.sync_copy(wbuf, acc.at[pl.ds(s * STRIPE, STRIPE)])
    pltpu.sync_copy(ones_hbm, ones_v)
    plsc.subcore_barrier()

    row0 = c * (R2 // 2) + s * DEG_ROWS_T

    def step(u, carry):
        r0 = row0 + u * K_DEG
        pltpu.sync_copy(dst_hbm.at[pl.ds(r0, K_DEG)], idx_v)
        for j in range(K_DEG):
            pltpu.sync_copy(ones_v, acc.at[idx_v.at[j]], add=True)
        return carry

    lax.fori_loop(0, DEG_SUPERS, step, 0)
    plsc.subcore_barrier()

    pltpu.sync_copy(acc.at[pl.ds(s * STRIPE, STRIPE)], wbuf)
    pltpu.sync_copy(wbuf, out_hbm.at[pl.ds(c * ACC_ROWS + s * STRIPE, STRIPE)])


@functools.partial(
    pl.kernel,
    mesh=_mesh,
    compiler_params=_sc_params,
    out_type=jax.ShapeDtypeStruct((ACC_ROWS, H), jnp.float32),
    scratch_types=[
        pltpu.VMEM((K, 128), jnp.int32),
        pltpu.VMEM((K, 128), jnp.int32),
        pltpu.VMEM((K, 128), jnp.int32),
        pltpu.VMEM((K, 128), jnp.int32),
        pltpu.VMEM((K, 128, HQ), jnp.float32),
        pltpu.VMEM((K, 128, HQ), jnp.float32),
        pltpu.VMEM((WCHUNK, HQ), jnp.float32),
        pltpu.VMEM_SHARED((ACC_ROWS, HQ), jnp.float32),
        pltpu.SemaphoreType.DMA,
        pltpu.SemaphoreType.DMA,
        pltpu.SemaphoreType.DMA,
        pltpu.SemaphoreType.DMA,
    ],
)
def _gs_kernel(t0_hbm, t1_hbm, t2_hbm, t3_hbm, src_hbm, dst_hbm, zeros_hbm,
               out_hbm,
               idx_sA, idx_dA, idx_sB, idx_dB, rowsA, rowsB, wbuf, acc,
               gsemA, gsemB, ssemA, ssemB):
    c = lax.axis_index("c")
    s = lax.axis_index("s")
    row0 = s * ROWS_T

    def run(q, ts_hbm):
        pltpu.sync_copy(zeros_hbm, wbuf)
        for m in range(STRIPE // WCHUNK):
            pltpu.sync_copy(wbuf, acc.at[pl.ds(s * STRIPE + m * WCHUNK, WCHUNK)])
        plsc.subcore_barrier()

        col = q * HQ

        def drain_gathers(rows, gsem):
            for j in range(K):
                pltpu.make_async_copy(
                    ts_hbm.at[pl.ds(0, 128)], rows.at[j], gsem).wait()

        def scatter(idx_d, rows, ssem):
            for j in range(K):
                pltpu.async_copy(rows.at[j], acc.at[idx_d.at[j]], ssem,
                                 add=True)

        def drain_scatters(idx_d, rows, ssem):
            for j in range(K):
                pltpu.make_async_copy(rows.at[j], acc.at[idx_d.at[j]],
                                      ssem).wait()

        def half(u, idx_s, idx_d, rows, gsem, ssem, pidx_d, prows, pgsem, pssem):
            @pl.when(u >= 2)
            def _():
                drain_scatters(idx_d, rows, ssem)

            r0 = row0 + u * K
            pltpu.sync_copy(src_hbm.at[pl.ds(r0, K)], idx_s)
            pltpu.sync_copy(dst_hbm.at[pl.ds(r0, K)], idx_d)
            for j in range(K):
                pltpu.async_copy(ts_hbm.at[idx_s.at[j]], rows.at[j], gsem)

            @pl.when(u >= 1)
            def _():
                drain_gathers(prows, pgsem)
                scatter(pidx_d, prows, pssem)

        def step(u, carry):
            @pl.when(u % 2 == 0)
            def _():
                half(u, idx_sA, idx_dA, rowsA, gsemA, ssemA,
                     idx_dB, rowsB, gsemB, ssemB)

            @pl.when(u % 2 == 1)
            def _():
                half(u, idx_sB, idx_dB, rowsB, gsemB, ssemB,
                     idx_dA, rowsA, gsemA, ssemA)

            return carry

        lax.fori_loop(0, SUPERS, step, 0)
        drain_gathers(rowsB, gsemB)
        scatter(idx_dB, rowsB, ssemB)
        drain_scatters(idx_dA, rowsA, ssemA)
        drain_scatters(idx_dB, rowsB, ssemB)
        plsc.subcore_barrier()
        for m in range(STRIPE // WCHUNK):
            off = s * STRIPE + m * WCHUNK
            pltpu.sync_copy(acc.at[pl.ds(off, WCHUNK)], wbuf)
            pltpu.sync_copy(wbuf, out_hbm.at[pl.ds(off, WCHUNK), pl.ds(col, HQ)])
        plsc.subcore_barrier()

    @pl.when(c == 0)
    def _():
        run(0, t0_hbm)
        run(1, t1_hbm)

    @pl.when(c == 1)
    def _():
        run(2, t2_hbm)
        run(3, t3_hbm)


def _quarters(t):
    return [t[:, q * HQ:(q + 1) * HQ] for q in range(4)]


def _t1_body(x_ref, deg_ref, embW_ref, embb_ref, W1_ref,
             t0_ref, t1_ref, t2_ref, t3_ref):
    h0 = jnp.dot(x_ref[...], embW_ref[...], preferred_element_type=jnp.float32)
    h0 = jnp.maximum(h0 + embb_ref[...][None, :], 0.0)
    dinv = lax.rsqrt(deg_ref[...] + 1.0)
    t = jnp.dot(h0 * dinv, W1_ref[...], preferred_element_type=jnp.float32)
    for ref, q in zip((t0_ref, t1_ref, t2_ref, t3_ref), _quarters(t)):
        ref[...] = q


_t1_call = pl.pallas_call(
    _t1_body,
    grid=(NB,),
    in_specs=[
        pl.BlockSpec((BR, F_IN), lambda i: (i, 0)),
        pl.BlockSpec((BR, 1), lambda i: (i, 0)),
        pl.BlockSpec((F_IN, H), lambda i: (0, 0)),
        pl.BlockSpec((H,), lambda i: (0,)),
        pl.BlockSpec((H, H), lambda i: (0, 0)),
    ],
    out_specs=[pl.BlockSpec((BR, HQ), lambda i: (i, 0))] * 4,
    out_shape=[jax.ShapeDtypeStruct((N, HQ), jnp.float32)] * 4,
)


def _t2a_body(s_ref, t0_ref, t1_ref, t2_ref, t3_ref, deg_ref, b_ref,
              z_ref, st_ref):
    i = pl.program_id(0)
    S = s_ref[...]
    ts = jnp.concatenate(
        [r[...] for r in (t0_ref, t1_ref, t2_ref, t3_ref)], axis=1)
    dinv = lax.rsqrt(deg_ref[...] + 1.0)
    z = dinv * (S + ts) + b_ref[...][None, :]
    z_ref[...] = z

    @pl.when(i == 0)
    def _():
        st_ref[...] = jnp.zeros_like(st_ref)

    rowid = i * BR + lax.broadcasted_iota(jnp.int32, (BR, 1), 0)
    zm = jnp.where(rowid < N, z, 0.0)
    st_ref[...] += jnp.stack([jnp.sum(zm, axis=0), jnp.sum(zm * zm, axis=0)])


_t2a_call = pl.pallas_call(
    _t2a_body,
    grid=(NB,),
    in_specs=[
        pl.BlockSpec((BR, H), lambda i: (i, 0)),
        pl.BlockSpec((BR, HQ), lambda i: (i, 0)),
        pl.BlockSpec((BR, HQ), lambda i: (i, 0)),
        pl.BlockSpec((BR, HQ), lambda i: (i, 0)),
        pl.BlockSpec((BR, HQ), lambda i: (i, 0)),
        pl.BlockSpec((BR, 1), lambda i: (i, 0)),
        pl.BlockSpec((H,), lambda i: (0,)),
    ],
    out_specs=[
        pl.BlockSpec((BR, H), lambda i: (i, 0)),
        pl.BlockSpec((2, H), lambda i: (0, 0)),
    ],
    out_shape=[
        jax.ShapeDtypeStruct((N, H), jnp.float32),
        jax.ShapeDtypeStruct((2, H), jnp.float32),
    ],
)


def _bn_relu(z, st, g, be):
    mu = st[0:1, :] * (1.0 / N)
    var = st[1:2, :] * (1.0 / N) - mu * mu
    rstd = lax.rsqrt(var + 1e-5)
    return jnp.maximum((z - mu) * rstd * g[None, :] + be[None, :], 0.0)


def _t2b_body(z_ref, st_ref, g_ref, be_ref, W_ref, deg_ref,
              t0_ref, t1_ref, t2_ref, t3_ref):
    h = _bn_relu(z_ref[...], st_ref[...], g_ref[...], be_ref[...])
    dinv = lax.rsqrt(deg_ref[...] + 1.0)
    t = jnp.dot(h * dinv, W_ref[...], preferred_element_type=jnp.float32)
    for ref, q in zip((t0_ref, t1_ref, t2_ref, t3_ref), _quarters(t)):
        ref[...] = q


_t2b_call = pl.pallas_call(
    _t2b_body,
    grid=(NB,),
    in_specs=[
        pl.BlockSpec((BR, H), lambda i: (i, 0)),
        pl.BlockSpec((2, H), lambda i: (0, 0)),
        pl.BlockSpec((H,), lambda i: (0,)),
        pl.BlockSpec((H,), lambda i: (0,)),
        pl.BlockSpec((H, H), lambda i: (0, 0)),
        pl.BlockSpec((BR, 1), lambda i: (i, 0)),
    ],
    out_specs=[pl.BlockSpec((BR, HQ), lambda i: (i, 0))] * 4,
    out_shape=[jax.ShapeDtypeStruct((N, HQ), jnp.float32)] * 4,
)


def _t3_body(z_ref, st_ref, g_ref, be_ref, clsW_ref, clsb_ref, out_ref):
    h = _bn_relu(z_ref[...], st_ref[...], g_ref[...], be_ref[...])
    out_ref[...] = (jnp.dot(h, clsW_ref[...], preferred_element_type=jnp.float32)
                    + clsb_ref[...][None, :])


_t3_call = pl.pallas_call(
    _t3_body,
    grid=(NB,),
    in_specs=[
        pl.BlockSpec((BR, H), lambda i: (i, 0)),
        pl.BlockSpec((2, H), lambda i: (0, 0)),
        pl.BlockSpec((H,), lambda i: (0,)),
        pl.BlockSpec((H,), lambda i: (0,)),
        pl.BlockSpec((H, C), lambda i: (0, 0)),
        pl.BlockSpec((C,), lambda i: (0,)),
    ],
    out_specs=pl.BlockSpec((BR, C), lambda i: (i, 0)),
    out_shape=jax.ShapeDtypeStruct((N, C), jnp.float32),
)


def kernel(x, edge_index, emb_W, emb_b, W1, b1, g1, be1, W2, b2, g2, be2, cls_W, cls_b):
    src = edge_index[0]
    dst = edge_index[1]
    pad = E_PAD - E
    pad_idx = jnp.arange(pad, dtype=jnp.int32)
    pad_src = (pad_idx * 97) % N
    pad_dst = N + (pad_idx % (ACC_ROWS - N))
    src2d = jnp.concatenate([src, pad_src]).reshape(R2, 128)
    dst2d = jnp.concatenate([dst, pad_dst]).reshape(R2, 128)

    zeros1d = jnp.zeros((STRIPE,), jnp.float32)
    zeros2d = jnp.zeros((WCHUNK, HQ), jnp.float32)
    ones128 = jnp.ones((128,), jnp.float32)

    degp = _deg_kernel(dst2d, zeros1d, ones128)
    deg_col = (degp[:N] + degp[ACC_ROWS:ACC_ROWS + N]).reshape(N, 1)

    ts1 = _t1_call(x, deg_col, emb_W, emb_b, W1)
    S1 = _gs_kernel(*ts1, src2d, dst2d, zeros2d)
    z1, st1 = _t2a_call(S1, *ts1, deg_col, b1)
    ts2 = _t2b_call(z1, st1, g1, be1, W2, deg_col)
    S2 = _gs_kernel(*ts2, src2d, dst2d, zeros2d)
    z2, st2 = _t2a_call(S2, *ts2, deg_col, b2)
    return _t3_call(z2, st2, g2, be2, cls_W, cls_b)
